# baseline (speedup 1.0000x reference)
import dataclasses
import functools

import jax
import jax.numpy as jnp
from jax import lax
from jax.experimental import pallas as pl
from jax.experimental.pallas import tpu as pltpu
from jax.experimental.pallas import tpu_sc as plsc

B, N, G = 64, 8192, 32 * 32 * 32
SIX_N = 6 * N
CHUNK = 2048


def _build_affine(output):
    planes = output[:, 0:3, :]
    nv = planes[..., 0:3]
    d = planes[..., 3]
    nn = jnp.sum(nv * nv, axis=-1)
    eye = jnp.eye(3, dtype=output.dtype)
    Mf = eye - 2.0 * nv[..., :, None] * nv[..., None, :] / nn[..., None, None]
    cf = -2.0 * d[..., None] * nv / nn[..., None]

    quats = output[:, 3:6, :]
    w = quats[..., 0]
    u = quats[..., 1:4]
    uu_sum = jnp.sum(u * u, axis=-1)
    nq = jnp.sqrt(w * w + uu_sum)
    uuT = u[..., :, None] * u[..., None, :]
    ux, uy, uz = u[..., 0], u[..., 1], u[..., 2]
    zz = jnp.zeros_like(ux)
    ucross = jnp.stack([
        jnp.stack([zz, -uz, uy], axis=-1),
        jnp.stack([uz, zz, -ux], axis=-1),
        jnp.stack([-uy, ux, zz], axis=-1),
    ], axis=-2)
    Mr = ((w * w - uu_sum)[..., None, None] * eye
          + 2.0 * uuT + 2.0 * w[..., None, None] * ucross) / nq[..., None, None]

    M = jnp.concatenate([Mf, Mr], axis=1)
    t = jnp.concatenate([cf, jnp.zeros_like(cf)], axis=1)
    Wfull = jnp.concatenate([M, t[..., None]], axis=-1)
    return Wfull.transpose(0, 2, 1, 3).reshape(B, 18, 4)


def _sym_planes(w_ref, pts_ref):
    wmat = w_ref[0]
    px = pts_ref[0, 0:1, :]
    py = pts_ref[0, 1:2, :]
    pz = pts_ref[0, 2:3, :]
    return (wmat[:, 0:1] * px + wmat[:, 1:2] * py
            + wmat[:, 2:3] * pz + wmat[:, 3:4])


def _lin_body(w_ref, pts_ref, lin_ref):
    sym = _sym_planes(w_ref, pts_ref)
    ix = jnp.floor(jnp.clip(sym[0:6], 0.0, 31.0)).astype(jnp.int32)
    iy = jnp.floor(jnp.clip(sym[6:12], 0.0, 31.0)).astype(jnp.int32)
    iz = jnp.floor(jnp.clip(sym[12:18], 0.0, 31.0)).astype(jnp.int32)
    lin = ix * 1024 + iy * 32 + iz
    lin_ref[0] = jnp.clip(lin, 0, G - 1)


_lin_call = pl.pallas_call(
    _lin_body,
    grid=(B,),
    in_specs=[
        pl.BlockSpec((1, 18, 4), lambda b: (b, 0, 0)),
        pl.BlockSpec((1, 3, N), lambda b: (b, 0, 0)),
    ],
    out_specs=pl.BlockSpec((1, 6, N), lambda b: (b, 0, 0)),
    out_shape=jax.ShapeDtypeStruct((B, 6, N), jnp.int32),
)


def _dist_body(w_ref, pts_ref, cp_ref, out_ref):
    b = pl.program_id(0)
    sym = _sym_planes(w_ref, pts_ref)
    dx = sym[0:6] - cp_ref[0, 0]
    dy = sym[6:12] - cp_ref[0, 1]
    dz = sym[12:18] - cp_ref[0, 2]
    dist = jnp.sqrt(dx * dx + dy * dy + dz * dz)
    part = jnp.sum(dist) * (1.0 / (N * B))

    @pl.when(b == 0)
    def _():
        out_ref[...] = jnp.zeros_like(out_ref)

    out_ref[...] += part


_dist_call = pl.pallas_call(
    _dist_body,
    grid=(B,),
    in_specs=[
        pl.BlockSpec((1, 18, 4), lambda b: (b, 0, 0)),
        pl.BlockSpec((1, 3, N), lambda b: (b, 0, 0)),
        pl.BlockSpec((1, 3, 6, N), lambda b: (b, 0, 0, 0)),
    ],
    out_specs=pl.BlockSpec((1, 1), lambda b: (0, 0)),
    out_shape=jax.ShapeDtypeStruct((1, 1), jnp.float32),
)


def _gather_body(closest_hbm, lin_hbm, cp_hbm, tbl, idxb, outx, outy, outz):
    wid = lax.axis_index("s") * 2 + lax.axis_index("c")

    for r in range(B // 32):
        b = wid * (B // 32) + r
        pltpu.sync_copy(closest_hbm.at[pl.ds(b * (3 * G), 3 * G)], tbl)

        @pl.loop(0, SIX_N // CHUNK)
        def _(ci):
            c0 = ci * CHUNK
            pltpu.sync_copy(lin_hbm.at[pl.ds(b * SIX_N + c0, CHUNK)], idxb)

            @pl.loop(0, CHUNK, step=16)
            def _(i):
                idx3 = idxb[pl.ds(i, 16)] * 3
                outx[pl.ds(i, 16)] = plsc.load_gather(tbl, [idx3])
                outy[pl.ds(i, 16)] = plsc.load_gather(tbl, [idx3 + 1])
                outz[pl.ds(i, 16)] = plsc.load_gather(tbl, [idx3 + 2])

            base = (b * 3) * SIX_N + c0
            pltpu.sync_copy(outx, cp_hbm.at[pl.ds(base, CHUNK)])
            pltpu.sync_copy(outy, cp_hbm.at[pl.ds(base + SIX_N, CHUNK)])
            pltpu.sync_copy(outz, cp_hbm.at[pl.ds(base + 2 * SIX_N, CHUNK)])


@functools.cache
def _gather_call():
    cparams = pltpu.CompilerParams()
    if "needs_layout_passes" in pltpu.CompilerParams.__dataclass_fields__:
        cparams = dataclasses.replace(cparams, needs_layout_passes=False)
    return pl.kernel(
        _gather_body,
        compiler_params=cparams,
        out_type=jax.ShapeDtypeStruct((B * 3 * SIX_N,), jnp.float32),
        mesh=plsc.VectorSubcoreMesh(core_axis_name="c", subcore_axis_name="s",
                                    num_cores=2, num_subcores=16),
        scratch_types=[
            pltpu.VMEM((3 * G,), jnp.float32),
            pltpu.VMEM((CHUNK,), jnp.int32),
            pltpu.VMEM((CHUNK,), jnp.float32),
            pltpu.VMEM((CHUNK,), jnp.float32),
            pltpu.VMEM((CHUNK,), jnp.float32),
        ],
    )


def kernel(output, points, closest):
    w = _build_affine(output)
    pts_t = points.transpose(0, 2, 1)
    lin = _lin_call(w, pts_t).reshape(B * SIX_N)
    cp = _gather_call()(closest.reshape(B * 3 * G), lin)
    loss = _dist_call(w, pts_t, cp.reshape(B, 3, 6, N))
    return loss.reshape(1)


# speedup vs baseline: 5.4629x; 5.4629x over previous
import dataclasses
import functools

import jax
import jax.numpy as jnp
from jax import lax
from jax.experimental import pallas as pl
from jax.experimental.pallas import tpu as pltpu
from jax.experimental.pallas import tpu_sc as plsc

B, N, G = 64, 8192, 32 * 32 * 32
SIX_N = 6 * N
CHUNK = 2048
GC = 2048


def _build_affine(output):
    planes = output[:, 0:3, :]
    nv = planes[..., 0:3]
    d = planes[..., 3]
    nn = jnp.sum(nv * nv, axis=-1)
    eye = jnp.eye(3, dtype=output.dtype)
    Mf = eye - 2.0 * nv[..., :, None] * nv[..., None, :] / nn[..., None, None]
    cf = -2.0 * d[..., None] * nv / nn[..., None]

    quats = output[:, 3:6, :]
    w = quats[..., 0]
    u = quats[..., 1:4]
    uu_sum = jnp.sum(u * u, axis=-1)
    nq = jnp.sqrt(w * w + uu_sum)
    uuT = u[..., :, None] * u[..., None, :]
    ux, uy, uz = u[..., 0], u[..., 1], u[..., 2]
    zz = jnp.zeros_like(ux)
    ucross = jnp.stack([
        jnp.stack([zz, -uz, uy], axis=-1),
        jnp.stack([uz, zz, -ux], axis=-1),
        jnp.stack([-uy, ux, zz], axis=-1),
    ], axis=-2)
    Mr = ((w * w - uu_sum)[..., None, None] * eye
          + 2.0 * uuT + 2.0 * w[..., None, None] * ucross) / nq[..., None, None]

    M = jnp.concatenate([Mf, Mr], axis=1)
    t = jnp.concatenate([cf, jnp.zeros_like(cf)], axis=1)
    return jnp.concatenate([M, t[..., None]], axis=-1)


def _repack_body(x_ref, y_ref, z_ref, tx_ref, ty_ref, tz_ref):
    tx_ref[...] = x_ref[0, 0]
    ty_ref[...] = y_ref[0, 0]
    tz_ref[...] = z_ref[0, 0]


_repack_call = pl.pallas_call(
    _repack_body,
    grid=(B, G // GC),
    in_specs=[pl.BlockSpec((1, 1, GC), lambda b, g: (b, 0, g))] * 3,
    out_specs=[pl.BlockSpec((GC,), lambda b, g: (b * (G // GC) + g,))] * 3,
    out_shape=[jax.ShapeDtypeStruct((B * G,), jnp.float32)] * 3,
)


def _sym_rows(w_ref, pts_ref):
    wv = w_ref[0, 0]
    px = pts_ref[0, 0:1, :]
    py = pts_ref[0, 1:2, :]
    pz = pts_ref[0, 2:3, :]
    sx = wv[0:1, 0:1] * px + wv[0:1, 1:2] * py + wv[0:1, 2:3] * pz + wv[0:1, 3:4]
    sy = wv[1:2, 0:1] * px + wv[1:2, 1:2] * py + wv[1:2, 2:3] * pz + wv[1:2, 3:4]
    sz = wv[2:3, 0:1] * px + wv[2:3, 1:2] * py + wv[2:3, 2:3] * pz + wv[2:3, 3:4]
    return sx, sy, sz


def _lin_body(w_ref, pts_ref, lin_ref):
    sx, sy, sz = _sym_rows(w_ref, pts_ref)
    ix = jnp.floor(jnp.clip(sx, 0.0, 31.0)).astype(jnp.int32)
    iy = jnp.floor(jnp.clip(sy, 0.0, 31.0)).astype(jnp.int32)
    iz = jnp.floor(jnp.clip(sz, 0.0, 31.0)).astype(jnp.int32)
    lin = jnp.clip(ix * 1024 + iy * 32 + iz, 0, G - 1)
    lin_ref[...] = lin.reshape(N)


_lin_call = pl.pallas_call(
    _lin_body,
    grid=(B, 6),
    in_specs=[
        pl.BlockSpec((1, 1, 3, 4), lambda b, s: (b, s, 0, 0)),
        pl.BlockSpec((1, 3, N), lambda b, s: (b, 0, 0)),
    ],
    out_specs=pl.BlockSpec((N,), lambda b, s: (b * 6 + s,)),
    out_shape=jax.ShapeDtypeStruct((B * SIX_N,), jnp.int32),
)


def _gather_body(tx_hbm, ty_hbm, tz_hbm, lin_hbm, cp_hbm,
                 tbx, tby, tbz, idxb, outx, outy, outz):
    wid = lax.axis_index("s") * 2 + lax.axis_index("c")

    for r in range(B // 32):
        b = wid * (B // 32) + r
        pltpu.sync_copy(tx_hbm.at[pl.ds(b * G, G)], tbx)
        pltpu.sync_copy(ty_hbm.at[pl.ds(b * G, G)], tby)
        pltpu.sync_copy(tz_hbm.at[pl.ds(b * G, G)], tbz)

        for s in range(6):
            @pl.loop(0, N // CHUNK)
            def _(ci):
                c0 = ci * CHUNK
                pltpu.sync_copy(
                    lin_hbm.at[pl.ds(b * SIX_N + s * N + c0, CHUNK)], idxb)

                @pl.loop(0, CHUNK, step=16)
                def _(i):
                    idxr = idxb[pl.ds(i, 16)]
                    outx[pl.ds(i, 16)] = plsc.load_gather(tbx, [idxr])
                    outy[pl.ds(i, 16)] = plsc.load_gather(tby, [idxr])
                    outz[pl.ds(i, 16)] = plsc.load_gather(tbz, [idxr])

                base = ((b * 6 + s) * 3) * N + c0
                pltpu.sync_copy(outx, cp_hbm.at[pl.ds(base, CHUNK)])
                pltpu.sync_copy(outy, cp_hbm.at[pl.ds(base + N, CHUNK)])
                pltpu.sync_copy(outz, cp_hbm.at[pl.ds(base + 2 * N, CHUNK)])


@functools.cache
def _gather_call():
    cparams = pltpu.CompilerParams()
    if "needs_layout_passes" in pltpu.CompilerParams.__dataclass_fields__:
        cparams = dataclasses.replace(cparams, needs_layout_passes=False)
    return pl.kernel(
        _gather_body,
        compiler_params=cparams,
        out_type=jax.ShapeDtypeStruct((B * 6 * 3 * N,), jnp.float32),
        mesh=plsc.VectorSubcoreMesh(core_axis_name="c", subcore_axis_name="s",
                                    num_cores=2, num_subcores=16),
        scratch_types=[
            pltpu.VMEM((G,), jnp.float32),
            pltpu.VMEM((G,), jnp.float32),
            pltpu.VMEM((G,), jnp.float32),
            pltpu.VMEM((CHUNK,), jnp.int32),
            pltpu.VMEM((CHUNK,), jnp.float32),
            pltpu.VMEM((CHUNK,), jnp.float32),
            pltpu.VMEM((CHUNK,), jnp.float32),
        ],
    )


def _dist_body(w_ref, pts_ref, cp_ref, out_ref):
    pid = pl.program_id(0) * 6 + pl.program_id(1)
    sx, sy, sz = _sym_rows(w_ref, pts_ref)
    cx = cp_ref[pl.ds(0, N)].reshape(1, N)
    cy = cp_ref[pl.ds(N, N)].reshape(1, N)
    cz = cp_ref[pl.ds(2 * N, N)].reshape(1, N)
    dx = sx - cx
    dy = sy - cy
    dz = sz - cz
    dist = jnp.sqrt(dx * dx + dy * dy + dz * dz)
    part = jnp.sum(dist) * (1.0 / (N * B))

    @pl.when(pid == 0)
    def _():
        out_ref[...] = jnp.zeros_like(out_ref)

    out_ref[...] += part


_dist_call = pl.pallas_call(
    _dist_body,
    grid=(B, 6),
    in_specs=[
        pl.BlockSpec((1, 1, 3, 4), lambda b, s: (b, s, 0, 0)),
        pl.BlockSpec((1, 3, N), lambda b, s: (b, 0, 0)),
        pl.BlockSpec((3 * N,), lambda b, s: (b * 6 + s,)),
    ],
    out_specs=pl.BlockSpec((1, 1), lambda b, s: (0, 0)),
    out_shape=jax.ShapeDtypeStruct((1, 1), jnp.float32),
)


def kernel(output, points, closest):
    w = _build_affine(output)
    pts_t = points.transpose(0, 2, 1)
    tx, ty, tz = _repack_call(closest[:, None, :, 0], closest[:, None, :, 1],
                              closest[:, None, :, 2])
    lin = _lin_call(w, pts_t)
    cp = _gather_call()(tx, ty, tz, lin)
    loss = _dist_call(w, pts_t, cp)
    return loss.reshape(1)


# speedup vs baseline: 14.9104x; 2.7294x over previous
import dataclasses
import functools

import jax
import jax.numpy as jnp
from jax import lax
from jax.experimental import pallas as pl
from jax.experimental.pallas import tpu as pltpu
from jax.experimental.pallas import tpu_sc as plsc

B, N, G = 64, 8192, 32 * 32 * 32
SIX_N = 6 * N
CHUNK = 2048


def _build_affine(output):
    planes = output[:, 0:3, :]
    nv = planes[..., 0:3]
    d = planes[..., 3]
    nn = jnp.sum(nv * nv, axis=-1)
    eye = jnp.eye(3, dtype=output.dtype)
    Mf = eye - 2.0 * nv[..., :, None] * nv[..., None, :] / nn[..., None, None]
    cf = -2.0 * d[..., None] * nv / nn[..., None]

    quats = output[:, 3:6, :]
    w = quats[..., 0]
    u = quats[..., 1:4]
    uu_sum = jnp.sum(u * u, axis=-1)
    nq = jnp.sqrt(w * w + uu_sum)
    uuT = u[..., :, None] * u[..., None, :]
    ux, uy, uz = u[..., 0], u[..., 1], u[..., 2]
    zz = jnp.zeros_like(ux)
    ucross = jnp.stack([
        jnp.stack([zz, -uz, uy], axis=-1),
        jnp.stack([uz, zz, -ux], axis=-1),
        jnp.stack([-uy, ux, zz], axis=-1),
    ], axis=-2)
    Mr = ((w * w - uu_sum)[..., None, None] * eye
          + 2.0 * uuT + 2.0 * w[..., None, None] * ucross) / nq[..., None, None]

    M = jnp.concatenate([Mf, Mr], axis=1)
    t = jnp.concatenate([cf, jnp.zeros_like(cf)], axis=1)
    Wfull = jnp.concatenate([M, t[..., None]], axis=-1)
    return Wfull.transpose(0, 2, 1, 3).reshape(B, 18, 4)


def _repack_body(x_ref, y_ref, z_ref, tx_ref, ty_ref, tz_ref):
    tx_ref[...] = x_ref[0, 0]
    ty_ref[...] = y_ref[0, 0]
    tz_ref[...] = z_ref[0, 0]


_repack_call = pl.pallas_call(
    _repack_body,
    grid=(B,),
    in_specs=[pl.BlockSpec((1, 1, G), lambda b: (b, 0, 0))] * 3,
    out_specs=[pl.BlockSpec((G,), lambda b: (b,))] * 3,
    out_shape=[jax.ShapeDtypeStruct((B * G,), jnp.float32)] * 3,
)


def _sym18(w_ref, pts_ref):
    wmat = w_ref[0]
    px = pts_ref[0, 0:1, :]
    py = pts_ref[0, 1:2, :]
    pz = pts_ref[0, 2:3, :]
    return (wmat[:, 0:1] * px + wmat[:, 1:2] * py
            + wmat[:, 2:3] * pz + wmat[:, 3:4])


def _lin_body(w_ref, pts_ref, *lin_refs):
    sym = _sym18(w_ref, pts_ref)
    ix = jnp.floor(jnp.clip(sym[0:6], 0.0, 31.0)).astype(jnp.int32)
    iy = jnp.floor(jnp.clip(sym[6:12], 0.0, 31.0)).astype(jnp.int32)
    iz = jnp.floor(jnp.clip(sym[12:18], 0.0, 31.0)).astype(jnp.int32)
    lin = jnp.clip(ix * 1024 + iy * 32 + iz, 0, G - 1)
    for s in range(6):
        lin_refs[s][...] = lin[s]


_lin_call = pl.pallas_call(
    _lin_body,
    grid=(B,),
    in_specs=[
        pl.BlockSpec((1, 18, 4), lambda b: (b, 0, 0)),
        pl.BlockSpec((1, 3, N), lambda b: (b, 0, 0)),
    ],
    out_specs=[pl.BlockSpec((N,), lambda b: (b,))] * 6,
    out_shape=[jax.ShapeDtypeStruct((B * N,), jnp.int32)] * 6,
)


def _gather_body(tx_hbm, ty_hbm, tz_hbm,
                 l0, l1, l2, l3, l4, l5, cp_hbm,
                 tbx, tby, tbz, idxb, outx, outy, outz):
    lin_refs = (l0, l1, l2, l3, l4, l5)
    wid = lax.axis_index("s") * 2 + lax.axis_index("c")

    for r in range(B // 32):
        b = wid * (B // 32) + r
        pltpu.sync_copy(tx_hbm.at[pl.ds(b * G, G)], tbx)
        pltpu.sync_copy(ty_hbm.at[pl.ds(b * G, G)], tby)
        pltpu.sync_copy(tz_hbm.at[pl.ds(b * G, G)], tbz)

        for s in range(6):
            @pl.loop(0, N // CHUNK)
            def _(ci):
                c0 = ci * CHUNK
                pltpu.sync_copy(lin_refs[s].at[pl.ds(b * N + c0, CHUNK)], idxb)

                @pl.loop(0, CHUNK, step=16)
                def _(i):
                    idxr = idxb[pl.ds(i, 16)]
                    outx[pl.ds(i, 16)] = plsc.load_gather(tbx, [idxr])
                    outy[pl.ds(i, 16)] = plsc.load_gather(tby, [idxr])
                    outz[pl.ds(i, 16)] = plsc.load_gather(tbz, [idxr])

                base = (b * 18 + s) * N + c0
                pltpu.sync_copy(outx, cp_hbm.at[pl.ds(base, CHUNK)])
                pltpu.sync_copy(outy, cp_hbm.at[pl.ds(base + 6 * N, CHUNK)])
                pltpu.sync_copy(outz, cp_hbm.at[pl.ds(base + 12 * N, CHUNK)])


@functools.cache
def _gather_call():
    cparams = pltpu.CompilerParams()
    if "needs_layout_passes" in pltpu.CompilerParams.__dataclass_fields__:
        cparams = dataclasses.replace(cparams, needs_layout_passes=False)
    return pl.kernel(
        _gather_body,
        compiler_params=cparams,
        out_type=jax.ShapeDtypeStruct((B * 18 * N,), jnp.float32),
        mesh=plsc.VectorSubcoreMesh(core_axis_name="c", subcore_axis_name="s",
                                    num_cores=2, num_subcores=16),
        scratch_types=[
            pltpu.VMEM((G,), jnp.float32),
            pltpu.VMEM((G,), jnp.float32),
            pltpu.VMEM((G,), jnp.float32),
            pltpu.VMEM((CHUNK,), jnp.int32),
            pltpu.VMEM((CHUNK,), jnp.float32),
            pltpu.VMEM((CHUNK,), jnp.float32),
            pltpu.VMEM((CHUNK,), jnp.float32),
        ],
    )


def _dist_body(w_ref, pts_ref, cp_ref, out_ref):
    b = pl.program_id(0)
    sym = _sym18(w_ref, pts_ref)
    cp = jnp.concatenate(
        [cp_ref[pl.ds(r * N, N)].reshape(1, N) for r in range(18)], axis=0)
    d = sym - cp
    sq = d * d
    ssq = sq[0:6] + sq[6:12] + sq[12:18]
    dist = jnp.sqrt(ssq)
    part = jnp.sum(dist) * (1.0 / (N * B))

    @pl.when(b == 0)
    def _():
        out_ref[...] = jnp.zeros_like(out_ref)

    out_ref[...] += part


_dist_call = pl.pallas_call(
    _dist_body,
    grid=(B,),
    in_specs=[
        pl.BlockSpec((1, 18, 4), lambda b: (b, 0, 0)),
        pl.BlockSpec((1, 3, N), lambda b: (b, 0, 0)),
        pl.BlockSpec((18 * N,), lambda b: (b,)),
    ],
    out_specs=pl.BlockSpec((1, 1), lambda b: (0, 0)),
    out_shape=jax.ShapeDtypeStruct((1, 1), jnp.float32),
)


def kernel(output, points, closest):
    w = _build_affine(output)
    pts_t = points.transpose(0, 2, 1)
    tx, ty, tz = _repack_call(closest[:, None, :, 0], closest[:, None, :, 1],
                              closest[:, None, :, 2])
    lins = _lin_call(w, pts_t)
    cp = _gather_call()(tx, ty, tz, *lins)
    loss = _dist_call(w, pts_t, cp)
    return loss.reshape(1)


# speedup vs baseline: 16.6620x; 1.1175x over previous
import dataclasses
import functools

import jax
import jax.numpy as jnp
from jax import lax
from jax.experimental import pallas as pl
from jax.experimental.pallas import tpu as pltpu
from jax.experimental.pallas import tpu_sc as plsc

B, N, G = 64, 8192, 32 * 32 * 32
SIX_N = 6 * N
CHUNK = 2048


def _build_affine(output):
    planes = output[:, 0:3, :]
    nv = planes[..., 0:3]
    d = planes[..., 3]
    nn = jnp.sum(nv * nv, axis=-1)
    eye = jnp.eye(3, dtype=output.dtype)
    Mf = eye - 2.0 * nv[..., :, None] * nv[..., None, :] / nn[..., None, None]
    cf = -2.0 * d[..., None] * nv / nn[..., None]

    quats = output[:, 3:6, :]
    w = quats[..., 0]
    u = quats[..., 1:4]
    uu_sum = jnp.sum(u * u, axis=-1)
    nq = jnp.sqrt(w * w + uu_sum)
    uuT = u[..., :, None] * u[..., None, :]
    ux, uy, uz = u[..., 0], u[..., 1], u[..., 2]
    zz = jnp.zeros_like(ux)
    ucross = jnp.stack([
        jnp.stack([zz, -uz, uy], axis=-1),
        jnp.stack([uz, zz, -ux], axis=-1),
        jnp.stack([-uy, ux, zz], axis=-1),
    ], axis=-2)
    Mr = ((w * w - uu_sum)[..., None, None] * eye
          + 2.0 * uuT + 2.0 * w[..., None, None] * ucross) / nq[..., None, None]

    M = jnp.concatenate([Mf, Mr], axis=1)
    t = jnp.concatenate([cf, jnp.zeros_like(cf)], axis=1)
    Wfull = jnp.concatenate([M, t[..., None]], axis=-1)
    return Wfull.transpose(0, 2, 1, 3).reshape(B, 18, 4)


def _repack_body(x_ref, y_ref, z_ref, tx_ref, ty_ref, tz_ref):
    tx_ref[...] = x_ref[0, 0]
    ty_ref[...] = y_ref[0, 0]
    tz_ref[...] = z_ref[0, 0]


_repack_call = pl.pallas_call(
    _repack_body,
    grid=(B,),
    in_specs=[pl.BlockSpec((1, 1, G), lambda b: (b, 0, 0))] * 3,
    out_specs=[pl.BlockSpec((G,), lambda b: (b,))] * 3,
    out_shape=[jax.ShapeDtypeStruct((B * G,), jnp.float32)] * 3,
)


def _sym18(w_ref, pts_ref):
    wmat = w_ref[0]
    px = pts_ref[0, 0:1, :]
    py = pts_ref[0, 1:2, :]
    pz = pts_ref[0, 2:3, :]
    return (wmat[:, 0:1] * px + wmat[:, 1:2] * py
            + wmat[:, 2:3] * pz + wmat[:, 3:4])


def _lin_body(w_ref, pts_ref, *lin_refs):
    sym = _sym18(w_ref, pts_ref)
    ix = jnp.floor(jnp.clip(sym[0:6], 0.0, 31.0)).astype(jnp.int32)
    iy = jnp.floor(jnp.clip(sym[6:12], 0.0, 31.0)).astype(jnp.int32)
    iz = jnp.floor(jnp.clip(sym[12:18], 0.0, 31.0)).astype(jnp.int32)
    lin = jnp.clip(ix * 1024 + iy * 32 + iz, 0, G - 1)
    for s in range(6):
        lin_refs[s][...] = lin[s]


_lin_call = pl.pallas_call(
    _lin_body,
    grid=(B,),
    in_specs=[
        pl.BlockSpec((1, 18, 4), lambda b: (b, 0, 0)),
        pl.BlockSpec((1, 3, N), lambda b: (b, 0, 0)),
    ],
    out_specs=[pl.BlockSpec((N,), lambda b: (b,))] * 6,
    out_shape=[jax.ShapeDtypeStruct((B * N,), jnp.int32)] * 6,
)


_NCH = SIX_N // CHUNK


def _gather_body(tx_hbm, ty_hbm, tz_hbm,
                 l0, l1, l2, l3, l4, l5, cp_hbm,
                 tbx, tby, tbz, idxb0, idxb1, ox0, oy0, oz0, ox1, oy1, oz1,
                 stbl, sidx0, sidx1, sout0, sout1):
    lin_refs = (l0, l1, l2, l3, l4, l5)
    idxbufs = (idxb0, idxb1)
    outbufs = ((ox0, oy0, oz0), (ox1, oy1, oz1))
    sidx = (sidx0, sidx1)
    sout = (sout0, sout1)
    wid = lax.axis_index("s") * 2 + lax.axis_index("c")

    def start_idx(b, k):
        s, co = divmod(k, N // CHUNK)
        return pltpu.async_copy(
            lin_refs[s].at[pl.ds(b * N + co * CHUNK, CHUNK)],
            idxbufs[k % 2], sidx[k % 2])

    def start_out(b, k):
        s, co = divmod(k, N // CHUNK)
        base = (b * 18 + s) * N + co * CHUNK
        bufs = outbufs[k % 2]
        return [pltpu.async_copy(bufs[c], cp_hbm.at[pl.ds(base + c * 6 * N, CHUNK)],
                                 sout[k % 2]) for c in range(3)]

    for r in range(B // 32):
        b = wid * (B // 32) + r
        tdescs = [pltpu.async_copy(tx_hbm.at[pl.ds(b * G, G)], tbx, stbl),
                  pltpu.async_copy(ty_hbm.at[pl.ds(b * G, G)], tby, stbl),
                  pltpu.async_copy(tz_hbm.at[pl.ds(b * G, G)], tbz, stbl)]
        idescs = [None] * _NCH
        odescs = [None] * _NCH
        idescs[0] = start_idx(b, 0)
        idescs[1] = start_idx(b, 1)
        for t in tdescs:
            t.wait()

        for k in range(_NCH):
            idescs[k].wait()
            if k >= 2:
                for o in odescs[k - 2]:
                    o.wait()
            ib = idxbufs[k % 2]
            ox, oy, oz = outbufs[k % 2]

            @pl.loop(0, CHUNK, step=16)
            def _(i):
                idxr = ib[pl.ds(i, 16)]
                ox[pl.ds(i, 16)] = plsc.load_gather(tbx, [idxr])
                oy[pl.ds(i, 16)] = plsc.load_gather(tby, [idxr])
                oz[pl.ds(i, 16)] = plsc.load_gather(tbz, [idxr])

            odescs[k] = start_out(b, k)
            if k + 2 < _NCH:
                idescs[k + 2] = start_idx(b, k + 2)

        for k in (_NCH - 2, _NCH - 1):
            for o in odescs[k]:
                o.wait()


@functools.cache
def _gather_call():
    cparams = pltpu.CompilerParams()
    if "needs_layout_passes" in pltpu.CompilerParams.__dataclass_fields__:
        cparams = dataclasses.replace(cparams, needs_layout_passes=False)
    return pl.kernel(
        _gather_body,
        compiler_params=cparams,
        out_type=jax.ShapeDtypeStruct((B * 18 * N,), jnp.float32),
        mesh=plsc.VectorSubcoreMesh(core_axis_name="c", subcore_axis_name="s",
                                    num_cores=2, num_subcores=16),
        scratch_types=[
            pltpu.VMEM((G,), jnp.float32),
            pltpu.VMEM((G,), jnp.float32),
            pltpu.VMEM((G,), jnp.float32),
            pltpu.VMEM((CHUNK,), jnp.int32),
            pltpu.VMEM((CHUNK,), jnp.int32),
            pltpu.VMEM((CHUNK,), jnp.float32),
            pltpu.VMEM((CHUNK,), jnp.float32),
            pltpu.VMEM((CHUNK,), jnp.float32),
            pltpu.VMEM((CHUNK,), jnp.float32),
            pltpu.VMEM((CHUNK,), jnp.float32),
            pltpu.VMEM((CHUNK,), jnp.float32),
            pltpu.SemaphoreType.DMA,
            pltpu.SemaphoreType.DMA,
            pltpu.SemaphoreType.DMA,
            pltpu.SemaphoreType.DMA,
            pltpu.SemaphoreType.DMA,
        ],
    )


def _dist_body(w_ref, pts_ref, cp_ref, out_ref):
    b = pl.program_id(0)
    sym = _sym18(w_ref, pts_ref)
    cp = jnp.concatenate(
        [cp_ref[pl.ds(r * N, N)].reshape(1, N) for r in range(18)], axis=0)
    d = sym - cp
    sq = d * d
    ssq = sq[0:6] + sq[6:12] + sq[12:18]
    dist = jnp.sqrt(ssq)
    part = jnp.sum(dist) * (1.0 / (N * B))

    @pl.when(b == 0)
    def _():
        out_ref[...] = jnp.zeros_like(out_ref)

    out_ref[...] += part


_dist_call = pl.pallas_call(
    _dist_body,
    grid=(B,),
    in_specs=[
        pl.BlockSpec((1, 18, 4), lambda b: (b, 0, 0)),
        pl.BlockSpec((1, 3, N), lambda b: (b, 0, 0)),
        pl.BlockSpec((18 * N,), lambda b: (b,)),
    ],
    out_specs=pl.BlockSpec((1, 1), lambda b: (0, 0)),
    out_shape=jax.ShapeDtypeStruct((1, 1), jnp.float32),
)


def kernel(output, points, closest):
    w = _build_affine(output)
    pts_t = points.transpose(0, 2, 1)
    tx, ty, tz = _repack_call(closest[:, None, :, 0], closest[:, None, :, 1],
                              closest[:, None, :, 2])
    lins = _lin_call(w, pts_t)
    cp = _gather_call()(tx, ty, tz, *lins)
    loss = _dist_call(w, pts_t, cp)
    return loss.reshape(1)
